# baseline (speedup 1.0000x reference)
import functools

import jax
import jax.numpy as jnp
from jax import lax
from jax.experimental import pallas as pl
from jax.experimental.pallas import tpu as pltpu
from jax.experimental.pallas import tpu_sc as plsc

_NUM_CORES = 2
_NUM_SUBCORES = 16
_NW = _NUM_CORES * _NUM_SUBCORES
_LANES = 128
_GRP = 16


def _make(B, D, V, b_per_w):
    mesh = plsc.VectorSubcoreMesh(core_axis_name="c", subcore_axis_name="s")
    n_grp = b_per_w // _GRP
    tail = V - _LANES
    main_lim = (V // _LANES) * _LANES
    c_max = V // _LANES - 1

    @functools.partial(
        pl.kernel,
        mesh=mesh,
        out_type=jax.ShapeDtypeStruct((D, B), jnp.float32),
        scratch_types=[
            pltpu.VMEM((b_per_w + _GRP,), jnp.int32),
            pltpu.VMEM((D, b_per_w), jnp.float32),
            pltpu.VMEM((_GRP, D, _LANES), jnp.float32),
            pltpu.VMEM((D, _LANES), jnp.float32),
            [pltpu.SemaphoreType.DMA] * _GRP,
        ],
        compiler_params=pltpu.CompilerParams(
            use_tc_tiling_on_sc=True, needs_layout_passes=False
        ),
    )
    def k(idx_hbm, tab_hbm, aux_hbm, out_hbm, idx_v, cols_v, bufs_v, aux_v,
          sems):
        wid = lax.axis_index("s") * _NUM_CORES + lax.axis_index("c")
        base = wid * b_per_w
        pltpu.sync_copy(idx_hbm.at[pl.ds(base, b_per_w)],
                        idx_v.at[pl.ds(0, b_per_w)])
        idx_v[pl.ds(b_per_w, _GRP)] = jnp.zeros((_GRP,), jnp.int32)
        pltpu.sync_copy(aux_hbm, aux_v)

        row_ids = lax.iota(jnp.int32, 16)

        def fire(s, c):
            pltpu.async_copy(
                tab_hbm.at[:, pl.ds(c * _LANES, _LANES)],
                bufs_v.at[s],
                sems[s],
            )

        def drain(s):
            pltpu.make_async_copy(
                tab_hbm.at[:, pl.ds(0, _LANES)], bufs_v.at[s], sems[s]
            ).wait()

        c0 = jnp.minimum(idx_v[pl.ds(0, _GRP)] >> 7, c_max)
        for s in range(_GRP):
            fire(s, c0[s])

        def body(g, carry):
            v = idx_v[pl.ds(g * _GRP, _GRP)]
            il_vec = v & (_LANES - 1)
            it_vec = jnp.maximum(v - tail, 0)
            vn = idx_v[pl.ds(g * _GRP + _GRP, _GRP)]
            cn = jnp.minimum(vn >> 7, c_max)
            for s in range(_GRP):
                drain(s)
                il = jnp.full((16,), il_vec[s], jnp.int32)
                col_main = plsc.load_gather(bufs_v.at[s], [row_ids, il])
                it = jnp.full((16,), it_vec[s], jnp.int32)
                col_tail = plsc.load_gather(aux_v, [row_ids, it])
                col = jnp.where(
                    jnp.full((16,), v[s], jnp.int32) < main_lim,
                    col_main,
                    col_tail,
                )
                plsc.store_scatter(
                    cols_v,
                    [row_ids, jnp.full((16,), g * _GRP + s, jnp.int32)],
                    col,
                )
                fire(s, cn[s])
            return carry

        lax.fori_loop(0, n_grp, body, 0)
        for s in range(_GRP):
            drain(s)
        pltpu.sync_copy(cols_v, out_hbm.at[:, pl.ds(base, b_per_w)])

    return k


def kernel(t, table):
    B = t.shape[0]
    V, D = table.shape
    b_per_w = B // _NW
    idx = t.astype(jnp.int32)
    aux = table[V - _LANES:, :].T
    out_t = _make(B, D, V, b_per_w)(idx, table.T, aux)
    return out_t.T


# speedup vs baseline: 1.0480x; 1.0480x over previous
import functools

import jax
import jax.numpy as jnp
from jax import lax
from jax.experimental import pallas as pl
from jax.experimental.pallas import tpu as pltpu
from jax.experimental.pallas import tpu_sc as plsc

_NUM_CORES = 2
_NUM_SUBCORES = 16
_NW = _NUM_CORES * _NUM_SUBCORES
_LANES = 128
_GRP = 16


def _make(B, D, V, b_per_w):
    mesh = plsc.VectorSubcoreMesh(core_axis_name="c", subcore_axis_name="s")
    n_grp = b_per_w // _GRP
    tail = V - _LANES
    main_lim = (V // _LANES) * _LANES
    c_max = V // _LANES - 1

    @functools.partial(
        pl.kernel,
        mesh=mesh,
        out_type=jax.ShapeDtypeStruct((D, B), jnp.float32),
        scratch_types=[
            pltpu.VMEM((b_per_w,), jnp.int32),
            pltpu.VMEM((D, b_per_w), jnp.float32),
            pltpu.VMEM((_GRP, D, _LANES), jnp.float32),
            pltpu.VMEM((D, _LANES), jnp.float32),
            [pltpu.SemaphoreType.DMA] * _GRP,
        ],
        compiler_params=pltpu.CompilerParams(
            use_tc_tiling_on_sc=True, needs_layout_passes=False
        ),
    )
    def k(idx_hbm, tab_hbm, aux_hbm, out_hbm, idx_v, cols_v, bufs_v, aux_v,
          sems):
        wid = lax.axis_index("s") * _NUM_CORES + lax.axis_index("c")
        base = wid * b_per_w
        pltpu.sync_copy(idx_hbm.at[pl.ds(base, b_per_w)], idx_v)
        pltpu.sync_copy(aux_hbm, aux_v)

        row_ids = lax.iota(jnp.int32, 16)

        def body(g, carry):
            v = idx_v[pl.ds(g * _GRP, _GRP)]
            c_vec = jnp.minimum(v >> 7, c_max)
            il_vec = v & (_LANES - 1)
            it_vec = jnp.maximum(v - tail, 0)
            for s in range(_GRP):
                pltpu.async_copy(
                    tab_hbm.at[:, pl.ds(c_vec[s] * _LANES, _LANES)],
                    bufs_v.at[s],
                    sems[s],
                )
            for s in range(_GRP):
                pltpu.make_async_copy(
                    tab_hbm.at[:, pl.ds(0, _LANES)],
                    bufs_v.at[s],
                    sems[s],
                ).wait()
                il = jnp.full((16,), il_vec[s], jnp.int32)
                col_main = plsc.load_gather(bufs_v.at[s], [row_ids, il])
                it = jnp.full((16,), it_vec[s], jnp.int32)
                col_tail = plsc.load_gather(aux_v, [row_ids, it])
                col = jnp.where(
                    jnp.full((16,), v[s], jnp.int32) < main_lim,
                    col_main,
                    col_tail,
                )
                plsc.store_scatter(
                    cols_v,
                    [row_ids, jnp.full((16,), g * _GRP + s, jnp.int32)],
                    col,
                )
            return carry

        lax.fori_loop(0, n_grp, body, 0)
        pltpu.sync_copy(cols_v, out_hbm.at[:, pl.ds(base, b_per_w)])

    return k


def kernel(t, table):
    B = t.shape[0]
    V, D = table.shape
    b_per_w = B // _NW
    idx = t.astype(jnp.int32)
    aux = table[V - _LANES:, :].T
    out_t = _make(B, D, V, b_per_w)(idx, table.T, aux)
    return out_t.T
